# Initial kernel scaffold; baseline (speedup 1.0000x reference)
import jax
import jax.numpy as jnp
from jax.experimental import pallas as pl


def kernel(input, configs_up, configs_down):
    raise NotImplementedError("write your pallas kernel here")


# speedup vs baseline: 1019.5343x; 1019.5343x over previous
import jax
import jax.numpy as jnp
from jax.experimental import pallas as pl
from jax.experimental.pallas import tpu as pltpu

_NE = 16
_NMO = 64
_NC = 128
_SB = 4


def _dets(rows, ri):
    cols = [rows[:, j * _NC:(j + 1) * _NC] for j in range(_NE)]
    det = None
    for k in range(_NE):
        colk = cols[k]
        xk = jnp.where(ri >= k, colk, 0.0)
        n2 = jnp.sum(xk * xk, axis=0, keepdims=True)
        norm = jnp.sqrt(n2)
        xkk = colk[k:k + 1, :]
        s = jnp.where(xkk >= 0.0, 1.0, -1.0)
        alpha = -s * norm
        det = alpha if det is None else det * alpha
        v = jnp.where(ri == k, xkk + s * norm, xk)
        vn2 = 2.0 * (n2 + norm * jnp.abs(xkk))
        beta = jnp.where(vn2 > 0.0, 2.0 / vn2, 0.0)
        for j in range(k + 1, _NE):
            w = jnp.sum(v * cols[j], axis=0, keepdims=True)
            cols[j] = cols[j] - v * (beta * w)
    return det


def _body(xu_ref, xd_ref, ohu_ref, ohd_ref, out_ref):
    ri = jax.lax.broadcasted_iota(jnp.int32, (_NE, _NC), 0)
    gu = jnp.dot(xu_ref[...], ohu_ref[...], preferred_element_type=jnp.float32,
                 precision=jax.lax.Precision.HIGHEST)
    gd = jnp.dot(xd_ref[...], ohd_ref[...], preferred_element_type=jnp.float32,
                 precision=jax.lax.Precision.HIGHEST)
    for s in range(_SB):
        du = _dets(gu[s * _NE:(s + 1) * _NE, :], ri)
        dd = _dets(gd[s * _NE:(s + 1) * _NE, :], ri)
        out_ref[0, s, :] = (du * dd)[0, :]


def kernel(input, configs_up, configs_down):
    S = input.shape[0]
    xu = input[:, :_NE, :].reshape(S * _NE, _NMO)
    xd = input[:, _NE:, :].reshape(S * _NE, _NMO)
    iota_m = jnp.arange(_NMO, dtype=configs_up.dtype)
    ohu = (configs_up.T[None, :, :] == iota_m[:, None, None]
           ).astype(jnp.float32).reshape(_NMO, _NE * _NC)
    ohd = (configs_down.T[None, :, :] == iota_m[:, None, None]
           ).astype(jnp.float32).reshape(_NMO, _NE * _NC)
    return pl.pallas_call(
        _body,
        grid=(S // _SB,),
        in_specs=[
            pl.BlockSpec((_SB * _NE, _NMO), lambda s: (s, 0)),
            pl.BlockSpec((_SB * _NE, _NMO), lambda s: (s, 0)),
            pl.BlockSpec((_NMO, _NE * _NC), lambda s: (0, 0)),
            pl.BlockSpec((_NMO, _NE * _NC), lambda s: (0, 0)),
        ],
        out_specs=pl.BlockSpec((1, _SB, _NC), lambda s: (s, 0, 0)),
        out_shape=jax.ShapeDtypeStruct((S // _SB, _SB, _NC), jnp.float32),
        compiler_params=pltpu.CompilerParams(
            dimension_semantics=("parallel",)),
    )(xu, xd, ohu, ohd).reshape(S, _NC)


# speedup vs baseline: 1147.8816x; 1.1259x over previous
import jax
import jax.numpy as jnp
from jax.experimental import pallas as pl
from jax.experimental.pallas import tpu as pltpu

_NE = 16
_NMO = 64
_NC = 128
_SB = 4


def _dets(rows, ri):
    cols = [rows[:, j * _NC:(j + 1) * _NC] for j in range(_NE)]
    det = None
    for k in range(_NE):
        colk = cols[k]
        xk = jnp.where(ri >= k, colk, 0.0)
        n2 = jnp.sum(xk * xk, axis=0, keepdims=True)
        norm = jnp.sqrt(n2)
        xkk = colk[k:k + 1, :]
        t = jnp.where(xkk >= 0.0, norm, -norm)
        det = -t if det is None else det * -t
        v = jnp.where(ri == k, xkk + t, xk)
        den = n2 + t * xkk
        beta = jnp.where(den > 0.0, 1.0 / den, 0.0)
        vb = beta * v
        for j in range(k + 1, _NE):
            w = jnp.sum(v * cols[j], axis=0, keepdims=True)
            cols[j] = cols[j] - vb * w
    return det


def _body(xu_ref, xd_ref, ohu_ref, ohd_ref, out_ref):
    ri = jax.lax.broadcasted_iota(jnp.int32, (_NE, _NC), 0)
    gu = jnp.dot(xu_ref[...], ohu_ref[...], preferred_element_type=jnp.float32,
                 precision=jax.lax.Precision.HIGHEST)
    gd = jnp.dot(xd_ref[...], ohd_ref[...], preferred_element_type=jnp.float32,
                 precision=jax.lax.Precision.HIGHEST)
    for s in range(_SB):
        du = _dets(gu[s * _NE:(s + 1) * _NE, :], ri)
        dd = _dets(gd[s * _NE:(s + 1) * _NE, :], ri)
        out_ref[0, s, :] = (du * dd)[0, :]


def kernel(input, configs_up, configs_down):
    S = input.shape[0]
    xu = input[:, :_NE, :].reshape(S * _NE, _NMO)
    xd = input[:, _NE:, :].reshape(S * _NE, _NMO)
    iota_m = jnp.arange(_NMO, dtype=configs_up.dtype)
    ohu = (configs_up.T[None, :, :] == iota_m[:, None, None]
           ).astype(jnp.float32).reshape(_NMO, _NE * _NC)
    ohd = (configs_down.T[None, :, :] == iota_m[:, None, None]
           ).astype(jnp.float32).reshape(_NMO, _NE * _NC)
    return pl.pallas_call(
        _body,
        grid=(S // _SB,),
        in_specs=[
            pl.BlockSpec((_SB * _NE, _NMO), lambda s: (s, 0)),
            pl.BlockSpec((_SB * _NE, _NMO), lambda s: (s, 0)),
            pl.BlockSpec((_NMO, _NE * _NC), lambda s: (0, 0)),
            pl.BlockSpec((_NMO, _NE * _NC), lambda s: (0, 0)),
        ],
        out_specs=pl.BlockSpec((1, _SB, _NC), lambda s: (s, 0, 0)),
        out_shape=jax.ShapeDtypeStruct((S // _SB, _SB, _NC), jnp.float32),
        compiler_params=pltpu.CompilerParams(
            dimension_semantics=("parallel",)),
    )(xu, xd, ohu, ohd).reshape(S, _NC)


# speedup vs baseline: 1176.0648x; 1.0246x over previous
import jax
import jax.numpy as jnp
from jax.experimental import pallas as pl
from jax.experimental.pallas import tpu as pltpu

_NE = 16
_NMO = 64
_NC = 128
_SB = 8


def _dets(rows, ri):
    cols = [rows[:, j * _NC:(j + 1) * _NC] for j in range(_NE)]
    det = None
    for k in range(_NE):
        colk = cols[k]
        xk = jnp.where(ri >= k, colk, 0.0)
        n2 = jnp.sum(xk * xk, axis=0, keepdims=True)
        norm = jnp.sqrt(n2)
        xkk = colk[k:k + 1, :]
        t = jnp.where(xkk >= 0.0, norm, -norm)
        det = -t if det is None else det * -t
        v = jnp.where(ri == k, xkk + t, xk)
        den = n2 + t * xkk
        beta = jnp.where(den > 0.0, 1.0 / den, 0.0)
        vb = beta * v
        for j in range(k + 1, _NE):
            w = jnp.sum(v * cols[j], axis=0, keepdims=True)
            cols[j] = cols[j] - vb * w
    return det


def _body(xu_ref, xd_ref, ohu_ref, ohd_ref, out_ref):
    ri = jax.lax.broadcasted_iota(jnp.int32, (_NE, _NC), 0)
    gu = jnp.dot(xu_ref[...], ohu_ref[...], preferred_element_type=jnp.float32,
                 precision=jax.lax.Precision.HIGHEST)
    gd = jnp.dot(xd_ref[...], ohd_ref[...], preferred_element_type=jnp.float32,
                 precision=jax.lax.Precision.HIGHEST)
    for s in range(_SB):
        du = _dets(gu[s * _NE:(s + 1) * _NE, :], ri)
        dd = _dets(gd[s * _NE:(s + 1) * _NE, :], ri)
        out_ref[0, s, :] = (du * dd)[0, :]


def kernel(input, configs_up, configs_down):
    S = input.shape[0]
    xu = input[:, :_NE, :].reshape(S * _NE, _NMO)
    xd = input[:, _NE:, :].reshape(S * _NE, _NMO)
    iota_m = jnp.arange(_NMO, dtype=configs_up.dtype)
    ohu = (configs_up.T[None, :, :] == iota_m[:, None, None]
           ).astype(jnp.float32).reshape(_NMO, _NE * _NC)
    ohd = (configs_down.T[None, :, :] == iota_m[:, None, None]
           ).astype(jnp.float32).reshape(_NMO, _NE * _NC)
    return pl.pallas_call(
        _body,
        grid=(S // _SB,),
        in_specs=[
            pl.BlockSpec((_SB * _NE, _NMO), lambda s: (s, 0)),
            pl.BlockSpec((_SB * _NE, _NMO), lambda s: (s, 0)),
            pl.BlockSpec((_NMO, _NE * _NC), lambda s: (0, 0)),
            pl.BlockSpec((_NMO, _NE * _NC), lambda s: (0, 0)),
        ],
        out_specs=pl.BlockSpec((1, _SB, _NC), lambda s: (s, 0, 0)),
        out_shape=jax.ShapeDtypeStruct((S // _SB, _SB, _NC), jnp.float32),
        compiler_params=pltpu.CompilerParams(
            dimension_semantics=("parallel",)),
    )(xu, xd, ohu, ohd).reshape(S, _NC)


# speedup vs baseline: 1226.6454x; 1.0430x over previous
import jax
import jax.numpy as jnp
from jax.experimental import pallas as pl
from jax.experimental.pallas import tpu as pltpu

_NE = 16
_NMO = 64
_NC = 128
_SB = 8


def _dets(rows, ri8):
    top = [rows[:8, j * _NC:(j + 1) * _NC] for j in range(_NE)]
    bot = [rows[8:, j * _NC:(j + 1) * _NC] for j in range(_NE)]
    det = None
    for k in range(_NE):
        if k < 8:
            xt = jnp.where(ri8 >= k, top[k], 0.0)
            xb = bot[k]
            n2 = jnp.sum(xt * xt + xb * xb, axis=0, keepdims=True)
            xkk = top[k][k:k + 1, :]
        else:
            xb = jnp.where(ri8 >= k - 8, bot[k], 0.0)
            n2 = jnp.sum(xb * xb, axis=0, keepdims=True)
            xkk = bot[k][k - 8:k - 7, :]
        norm = jnp.sqrt(n2)
        t = jnp.where(xkk >= 0.0, norm, -norm)
        det = -t if det is None else det * -t
        den = n2 + t * xkk
        beta = jnp.where(den > 0.0, 1.0 / den, 0.0)
        if k < 8:
            vt = jnp.where(ri8 == k, xkk + t, xt)
            vbt = beta * vt
            vbb = beta * xb
            for j in range(k + 1, _NE):
                w = jnp.sum(vt * top[j] + xb * bot[j], axis=0, keepdims=True)
                top[j] = top[j] - vbt * w
                bot[j] = bot[j] - vbb * w
        else:
            vb_ = jnp.where(ri8 == k - 8, xkk + t, xb)
            vbb = beta * vb_
            for j in range(k + 1, _NE):
                w = jnp.sum(vb_ * bot[j], axis=0, keepdims=True)
                bot[j] = bot[j] - vbb * w
    return det


def _body(xu_ref, xd_ref, ohu_ref, ohd_ref, out_ref):
    ri = jax.lax.broadcasted_iota(jnp.int32, (8, _NC), 0)
    gu = jnp.dot(xu_ref[...], ohu_ref[...], preferred_element_type=jnp.float32,
                 precision=jax.lax.Precision.HIGHEST)
    gd = jnp.dot(xd_ref[...], ohd_ref[...], preferred_element_type=jnp.float32,
                 precision=jax.lax.Precision.HIGHEST)
    for s in range(_SB):
        du = _dets(gu[s * _NE:(s + 1) * _NE, :], ri)
        dd = _dets(gd[s * _NE:(s + 1) * _NE, :], ri)
        out_ref[0, s, :] = (du * dd)[0, :]


def kernel(input, configs_up, configs_down):
    S = input.shape[0]
    xu = input[:, :_NE, :].reshape(S * _NE, _NMO)
    xd = input[:, _NE:, :].reshape(S * _NE, _NMO)
    iota_m = jnp.arange(_NMO, dtype=configs_up.dtype)
    ohu = (configs_up.T[None, :, :] == iota_m[:, None, None]
           ).astype(jnp.float32).reshape(_NMO, _NE * _NC)
    ohd = (configs_down.T[None, :, :] == iota_m[:, None, None]
           ).astype(jnp.float32).reshape(_NMO, _NE * _NC)
    return pl.pallas_call(
        _body,
        grid=(S // _SB,),
        in_specs=[
            pl.BlockSpec((_SB * _NE, _NMO), lambda s: (s, 0)),
            pl.BlockSpec((_SB * _NE, _NMO), lambda s: (s, 0)),
            pl.BlockSpec((_NMO, _NE * _NC), lambda s: (0, 0)),
            pl.BlockSpec((_NMO, _NE * _NC), lambda s: (0, 0)),
        ],
        out_specs=pl.BlockSpec((1, _SB, _NC), lambda s: (s, 0, 0)),
        out_shape=jax.ShapeDtypeStruct((S // _SB, _SB, _NC), jnp.float32),
        compiler_params=pltpu.CompilerParams(
            dimension_semantics=("parallel",)),
    )(xu, xd, ohu, ohd).reshape(S, _NC)
